# baseline (speedup 1.0000x reference)
import jax
import jax.numpy as jnp
from jax.experimental import pallas as pl


def _pred_body(hg_ref, w_ref, b_ref, o_ref):
    o_ref[...] = hg_ref[...] @ w_ref[...] + b_ref[...]


def _batch_norm(x, g, b):
    mu = jnp.mean(x, axis=0)
    var = jnp.var(x, axis=0)
    return (x - mu) / jnp.sqrt(var + 1e-5) * g + b


def _gen_conv(h, src, dst, node_basis, edge_basis, W, b, t, N):
    m = jax.nn.relu(h[src] * node_basis[src] + edge_basis) + 1e-7
    logits = m * t
    mx = jax.ops.segment_max(logits, dst, num_segments=N)
    ex = jnp.exp(logits - mx[dst])
    denom = jax.ops.segment_sum(ex, dst, num_segments=N)
    alpha = ex / (denom[dst] + 1e-16)
    aggr = jax.ops.segment_sum(m * alpha, dst, num_segments=N)
    return (h + aggr) @ W + b


def kernel(x_g, edge_index_g, edge_attr_g, x_lg, edge_index_lg, edge_dist_basis,
           edge_attr_lg, batch, W_enc, b_enc, W_msg, b_msg, W_nb, b_nb, W_eb, b_eb,
           W_mlp, b_mlp, gamma, beta, t, W_pred, b_pred):
    N_G = x_g.shape[0]
    L = W_mlp.shape[0]
    NUM_GRAPHS = 100
    h0 = x_g @ W_enc + b_enc
    src_g, dst_g = edge_index_g[0], edge_index_g[1]
    msg1 = jnp.take(h0, src_g, axis=0)
    msg2 = jnp.take(h0, dst_g, axis=0)
    msg_concat = jnp.concatenate([msg1, msg2, edge_attr_g, x_lg], axis=-1)
    h = msg_concat @ W_msg + b_msg
    node_basis = edge_dist_basis @ W_nb + b_nb
    edge_basis = edge_attr_lg @ W_eb + b_eb
    src, dst = edge_index_lg[0], edge_index_lg[1]
    N = h.shape[0]
    h = _gen_conv(h, src, dst, node_basis, edge_basis, W_mlp[0], b_mlp[0], t[0], N)
    for layer in range(1, L):
        h1 = _batch_norm(h, gamma[layer - 1], beta[layer - 1])
        h2 = jax.nn.relu(h1)
        h = _gen_conv(h2, src, dst, node_basis, edge_basis, W_mlp[layer], b_mlp[layer], t[layer], N) + h
    h = _batch_norm(h, gamma[L - 1], beta[L - 1])
    final_node_emb = jax.ops.segment_sum(h, dst_g, num_segments=N_G)
    sums = jax.ops.segment_sum(final_node_emb, batch, num_segments=NUM_GRAPHS)
    counts = jax.ops.segment_sum(jnp.ones((N_G,), jnp.float32), batch, num_segments=NUM_GRAPHS)
    h_graph = sums / jnp.maximum(counts, 1.0)[:, None]
    out = pl.pallas_call(
        _pred_body,
        out_shape=jax.ShapeDtypeStruct((NUM_GRAPHS, W_pred.shape[1]), jnp.float32),
    )(h_graph, W_pred, b_pred[None, :])
    return out


# speedup vs baseline: 1.9127x; 1.9127x over previous
import functools

import jax
import jax.numpy as jnp
from jax import lax
from jax.experimental import pallas as pl
from jax.experimental.pallas import tpu as pltpu
from jax.experimental.pallas import tpu_sc as plsc

N_G = 10000
E_G = 160000
E_LG = 480000
HID = 128
NUM_GRAPHS = 100
NREG = HID // 16

NC, NS = 2, 16
NW = NC * NS

D_BLK = 200
NBLK = E_G // D_BLK
BPW = NBLK // NW
E_CHK = 128
EP = 481280
BOFF_PAD = 832

NCH_G = E_G // E_CHK


def _mesh():
    return plsc.VectorSubcoreMesh(core_axis_name="c", subcore_axis_name="s")


def _wid():
    return lax.axis_index("s") * NC + lax.axis_index("c")


def _edge_pass_body(u_h, eb_h, sp_h, dl_h, bo_h, tl_h, nd_h,
                    bo_v, t_v, idx_v, dl_v, u_v, eb_v, acc, sem_u, sem_eb):
    wid = _wid()
    pltpu.sync_copy(bo_h, bo_v)
    pltpu.sync_copy(tl_h, t_v)
    tvec = t_v[...]

    def block_fn(bi, carry):
        b = wid * BPW + bi
        bvec = bo_v[pl.ds(b, 16)]
        e0 = bvec[0]
        e1 = bvec[1]

        def zero_fn(d, c2):
            for j in range(2 * NREG):
                acc[d, pl.ds(j * 16, 16)] = jnp.zeros((16,), jnp.float32)
            return c2
        lax.fori_loop(0, D_BLK, zero_fn, 0)

        c0 = (e0 // 8) * 8
        nch = (e1 - c0 + E_CHK - 1) // E_CHK

        def chunk_fn(k, c2):
            c = c0 + k * E_CHK
            pltpu.sync_copy(sp_h.at[pl.ds(c, E_CHK)], idx_v)
            pltpu.sync_copy(dl_h.at[pl.ds(c, E_CHK)], dl_v.at[pl.ds(0, E_CHK)])
            cp_e = pltpu.async_copy(eb_h.at[pl.ds(c, E_CHK)], eb_v, sem_eb)
            cp_u = pltpu.async_copy(u_h.at[idx_v], u_v, sem_u)
            cp_e.wait()
            cp_u.wait()
            lo = jnp.maximum(e0 - c, 0)
            hi = jnp.minimum(e1 - c, E_CHK)

            def edge_fn(e, c3):
                d = dl_v[pl.ds(e, 16)][0]
                for r in range(NREG):
                    uv = u_v[e, pl.ds(r * 16, 16)]
                    ev = eb_v[e, pl.ds(r * 16, 16)]
                    m = jnp.maximum(uv + ev, 0.0) + 1e-7
                    ex = jnp.exp(m * tvec)
                    plsc.addupdate(acc.at[d, pl.ds(r * 16, 16)], m * ex)
                    plsc.addupdate(acc.at[d, pl.ds(HID + r * 16, 16)], ex)
                return c3
            lax.fori_loop(lo, hi, edge_fn, 0)
            return c2
        lax.fori_loop(0, nch, chunk_fn, 0)
        pltpu.sync_copy(acc, nd_h.at[pl.ds(b * D_BLK, D_BLK)])
        return carry
    lax.fori_loop(0, BPW, block_fn, 0)


@functools.partial(
    pl.kernel,
    out_type=jax.ShapeDtypeStruct((E_G, 2 * HID), jnp.float32),
    mesh=_mesh(),
    scratch_types=[
        pltpu.VMEM((BOFF_PAD,), jnp.int32),
        pltpu.VMEM((16,), jnp.float32),
        pltpu.VMEM((E_CHK,), jnp.int32),
        pltpu.VMEM((E_CHK + 16,), jnp.int32),
        pltpu.VMEM((E_CHK, HID), jnp.float32),
        pltpu.VMEM((E_CHK, HID), jnp.float32),
        pltpu.VMEM((D_BLK, 2 * HID), jnp.float32),
        pltpu.SemaphoreType.DMA,
        pltpu.SemaphoreType.DMA,
    ],
)
def _edge_pass(*refs):
    _edge_pass_body(*refs)


def _msg_body(a_h, b_h, c_h, nb_h, sg_h, dg_h, hm_h, u0_h,
              sgv, dgv, av, bv, cv, nv, s1, s2, s3, s4):
    wid = _wid()

    def chunk_fn(k, carry):
        ch = wid + NW * k

        @pl.when(ch < NCH_G)
        def _():
            base = ch * E_CHK
            pltpu.sync_copy(sg_h.at[pl.ds(base, E_CHK)], sgv)
            pltpu.sync_copy(dg_h.at[pl.ds(base, E_CHK)], dgv)
            cp1 = pltpu.async_copy(a_h.at[sgv], av, s1)
            cp2 = pltpu.async_copy(b_h.at[dgv], bv, s2)
            cp3 = pltpu.async_copy(c_h.at[pl.ds(base, E_CHK)], cv, s3)
            cp4 = pltpu.async_copy(nb_h.at[pl.ds(base, E_CHK)], nv, s4)
            cp1.wait()
            cp2.wait()
            cp3.wait()
            cp4.wait()

            def edge_fn(e, c3):
                for r in range(NREG):
                    sl = pl.ds(r * 16, 16)
                    hm = av[e, sl] + bv[e, sl] + cv[e, sl]
                    av[e, sl] = hm
                    cv[e, sl] = hm * nv[e, sl]
                return c3
            lax.fori_loop(0, E_CHK, edge_fn, 0)
            pltpu.sync_copy(av, hm_h.at[pl.ds(base, E_CHK)])
            pltpu.sync_copy(cv, u0_h.at[pl.ds(base, E_CHK)])
        return carry
    lax.fori_loop(0, (NCH_G + NW - 1) // NW, chunk_fn, 0)


@functools.partial(
    pl.kernel,
    out_type=[jax.ShapeDtypeStruct((E_G, HID), jnp.float32),
              jax.ShapeDtypeStruct((E_G, HID), jnp.float32)],
    mesh=_mesh(),
    scratch_types=[
        pltpu.VMEM((E_CHK,), jnp.int32),
        pltpu.VMEM((E_CHK,), jnp.int32),
        pltpu.VMEM((E_CHK, HID), jnp.float32),
        pltpu.VMEM((E_CHK, HID), jnp.float32),
        pltpu.VMEM((E_CHK, HID), jnp.float32),
        pltpu.VMEM((E_CHK, HID), jnp.float32),
        pltpu.SemaphoreType.DMA,
        pltpu.SemaphoreType.DMA,
        pltpu.SemaphoreType.DMA,
        pltpu.SemaphoreType.DMA,
    ],
)
def _msg_gather(*refs):
    _msg_body(*refs)


def _readout_body(hf_h, ge_h, part_h, gev, hv, acc, s1):
    wid = _wid()

    def zero_fn(d, c2):
        for j in range(NREG):
            acc[d, pl.ds(j * 16, 16)] = jnp.zeros((16,), jnp.float32)
        return c2
    lax.fori_loop(0, NUM_GRAPHS, zero_fn, 0)

    def chunk_fn(k, carry):
        ch = wid + NW * k

        @pl.when(ch < NCH_G)
        def _():
            base = ch * E_CHK
            pltpu.sync_copy(ge_h.at[pl.ds(base, E_CHK)], gev.at[pl.ds(0, E_CHK)])
            pltpu.sync_copy(hf_h.at[pl.ds(base, E_CHK)], hv)

            def edge_fn(e, c3):
                g = gev[pl.ds(e, 16)][0]
                for r in range(NREG):
                    plsc.addupdate(acc.at[g, pl.ds(r * 16, 16)], hv[e, pl.ds(r * 16, 16)])
                return c3
            lax.fori_loop(0, E_CHK, edge_fn, 0)
        return carry
    lax.fori_loop(0, (NCH_G + NW - 1) // NW, chunk_fn, 0)
    pltpu.sync_copy(acc, part_h.at[wid])


@functools.partial(
    pl.kernel,
    out_type=jax.ShapeDtypeStruct((NW, NUM_GRAPHS, HID), jnp.float32),
    mesh=_mesh(),
    scratch_types=[
        pltpu.VMEM((E_CHK + 16,), jnp.int32),
        pltpu.VMEM((E_CHK, HID), jnp.float32),
        pltpu.VMEM((NUM_GRAPHS, HID), jnp.float32),
        pltpu.SemaphoreType.DMA,
    ],
)
def _readout(*refs):
    _readout_body(*refs)


def _enc_body(x_ref, we_ref, be_ref, w1_ref, w2_ref, a_ref, b_ref):
    h0 = jnp.dot(x_ref[...], we_ref[...], preferred_element_type=jnp.float32) + be_ref[...]
    a_ref[...] = jnp.dot(h0, w1_ref[...], preferred_element_type=jnp.float32)
    b_ref[...] = jnp.dot(h0, w2_ref[...], preferred_element_type=jnp.float32)


def _encoder(x_g, W_enc, b_enc, Wm1, Wm2):
    return pl.pallas_call(
        _enc_body,
        out_shape=[jax.ShapeDtypeStruct((N_G, HID), jnp.float32),
                   jax.ShapeDtypeStruct((N_G, HID), jnp.float32)],
    )(x_g, W_enc, b_enc, Wm1, Wm2)


def _smallk_mm(x_ref, w_ref, k):
    out = x_ref[:, 0:1] * w_ref[0:1, :]
    for j in range(1, k):
        out = out + x_ref[:, j:j + 1] * w_ref[j:j + 1, :]
    return out


def _cnb_body(ea_ref, xlg_ref, edb_ref, w3_ref, w4_ref, bm_ref, wnb_ref, bnb_ref,
              c_ref, nb_ref):
    c_ref[...] = (_smallk_mm(ea_ref, w3_ref, 16) + _smallk_mm(xlg_ref, w4_ref, 4)
                  + bm_ref[...])
    nb_ref[...] = _smallk_mm(edb_ref, wnb_ref, 4) + bnb_ref[...]


def _c_and_nb(ea_g, x_lg, edb, Wm3, Wm4, b_msg, W_nb, b_nb):
    R = 4000
    grid = (E_G // R,)
    return pl.pallas_call(
        _cnb_body,
        grid=grid,
        in_specs=[
            pl.BlockSpec((R, 16), lambda i: (i, 0)),
            pl.BlockSpec((R, 4), lambda i: (i, 0)),
            pl.BlockSpec((R, 4), lambda i: (i, 0)),
            pl.BlockSpec((16, HID), lambda i: (0, 0)),
            pl.BlockSpec((4, HID), lambda i: (0, 0)),
            pl.BlockSpec((1, HID), lambda i: (0, 0)),
            pl.BlockSpec((4, HID), lambda i: (0, 0)),
            pl.BlockSpec((1, HID), lambda i: (0, 0)),
        ],
        out_specs=[pl.BlockSpec((R, HID), lambda i: (i, 0)),
                   pl.BlockSpec((R, HID), lambda i: (i, 0))],
        out_shape=[jax.ShapeDtypeStruct((E_G, HID), jnp.float32),
                   jax.ShapeDtypeStruct((E_G, HID), jnp.float32)],
    )(ea_g, x_lg, edb, Wm3, Wm4, b_msg, W_nb, b_nb)


def _eb_body(ea_ref, web_ref, beb_ref, o_ref):
    o_ref[...] = _smallk_mm(ea_ref, web_ref, 4) + beb_ref[...]


def _edge_basis(ea_lg_p, W_eb, b_eb):
    R = 2560
    assert EP % R == 0
    grid = (EP // R,)
    return pl.pallas_call(
        _eb_body,
        grid=grid,
        in_specs=[
            pl.BlockSpec((R, 4), lambda i: (i, 0)),
            pl.BlockSpec((4, HID), lambda i: (0, 0)),
            pl.BlockSpec((1, HID), lambda i: (0, 0)),
        ],
        out_specs=pl.BlockSpec((R, HID), lambda i: (i, 0)),
        out_shape=jax.ShapeDtypeStruct((EP, HID), jnp.float32),
    )(ea_lg_p, W_eb, b_eb)


def _combine_body(has_res, nsteps, x2_ref, nd_ref, res_ref, w_ref, b_ref,
                  h_ref, st_ref, acc_ref):
    i = pl.program_id(0)
    num = nd_ref[:, :HID]
    den = nd_ref[:, HID:]
    xa = x2_ref[...] + num / (den + 1e-16)
    h = jnp.dot(xa, w_ref[...], preferred_element_type=jnp.float32) + b_ref[...]
    if has_res:
        h = h + res_ref[...]
    h_ref[...] = h

    @pl.when(i == 0)
    def _():
        acc_ref[...] = jnp.zeros_like(acc_ref)
    s = jnp.sum(h, axis=0, keepdims=True)
    s2 = jnp.sum(h * h, axis=0, keepdims=True)
    acc_ref[0:1, :] += s
    acc_ref[1:2, :] += s2

    @pl.when(i == nsteps - 1)
    def _():
        st_ref[...] = acc_ref[...]


def _combine(x2, nd, res, W, b):
    R = 2000
    nsteps = E_G // R
    has_res = res is not None
    in_specs = [
        pl.BlockSpec((R, HID), lambda i: (i, 0)),
        pl.BlockSpec((R, 2 * HID), lambda i: (i, 0)),
    ]
    args = [x2, nd]
    if has_res:
        in_specs.append(pl.BlockSpec((R, HID), lambda i: (i, 0)))
        args.append(res)
    in_specs += [pl.BlockSpec((HID, HID), lambda i: (0, 0)),
                 pl.BlockSpec((1, HID), lambda i: (0, 0))]
    args += [W, b]

    def body(*refs):
        if has_res:
            x2_ref, nd_ref, res_ref, w_ref, b_ref, h_ref, st_ref, acc_ref = refs
        else:
            x2_ref, nd_ref, w_ref, b_ref, h_ref, st_ref, acc_ref = refs
            res_ref = None
        _combine_body(has_res, nsteps, x2_ref, nd_ref, res_ref, w_ref, b_ref,
                      h_ref, st_ref, acc_ref)

    return pl.pallas_call(
        body,
        grid=(nsteps,),
        in_specs=in_specs,
        out_specs=[pl.BlockSpec((R, HID), lambda i: (i, 0)),
                   pl.BlockSpec((8, HID), lambda i: (0, 0))],
        out_shape=[jax.ShapeDtypeStruct((E_G, HID), jnp.float32),
                   jax.ShapeDtypeStruct((8, HID), jnp.float32)],
        scratch_shapes=[pltpu.VMEM((8, HID), jnp.float32)],
    )(*args)


def _norm_body(with_relu_u, h_ref, st_ref, nb_ref, g_ref, bt_ref, h2_ref, u_ref):
    mu = st_ref[0:1, :] / E_G
    var = st_ref[1:2, :] / E_G - mu * mu
    inv = lax.rsqrt(var + 1e-5)
    xn = (h_ref[...] - mu) * inv * g_ref[...] + bt_ref[...]
    if with_relu_u:
        h2 = jnp.maximum(xn, 0.0)
        h2_ref[...] = h2
        u_ref[...] = h2 * nb_ref[...]
    else:
        h2_ref[...] = xn


def _norm(h, st, nb, g, bt, with_relu_u):
    R = 2000
    nsteps = E_G // R
    in_specs = [
        pl.BlockSpec((R, HID), lambda i: (i, 0)),
        pl.BlockSpec((8, HID), lambda i: (0, 0)),
        pl.BlockSpec((R, HID), lambda i: (i, 0)),
        pl.BlockSpec((1, HID), lambda i: (0, 0)),
        pl.BlockSpec((1, HID), lambda i: (0, 0)),
    ]
    if with_relu_u:
        out_specs = [pl.BlockSpec((R, HID), lambda i: (i, 0)),
                     pl.BlockSpec((R, HID), lambda i: (i, 0))]
        out_shape = [jax.ShapeDtypeStruct((E_G, HID), jnp.float32),
                     jax.ShapeDtypeStruct((E_G, HID), jnp.float32)]
    else:
        out_specs = [pl.BlockSpec((R, HID), lambda i: (i, 0))]
        out_shape = [jax.ShapeDtypeStruct((E_G, HID), jnp.float32)]

    def body(*refs):
        if with_relu_u:
            h_ref, st_ref, nb_ref, g_ref, bt_ref, h2_ref, u_ref = refs
        else:
            h_ref, st_ref, nb_ref, g_ref, bt_ref, h2_ref = refs
            u_ref = None
        _norm_body(with_relu_u, h_ref, st_ref, nb_ref, g_ref, bt_ref, h2_ref, u_ref)

    res = pl.pallas_call(
        body,
        grid=(nsteps,),
        in_specs=in_specs,
        out_specs=out_specs,
        out_shape=out_shape,
    )(h, st, nb, g, bt)
    return res if with_relu_u else res[0]


def _final_body(p_ref, cnt_ref, wp_ref, bp_ref, o_ref):
    s = p_ref[0]
    for w in range(1, NW):
        s = s + p_ref[w]
    hg = s / cnt_ref[...]
    o_ref[...] = jnp.dot(hg, wp_ref[...], preferred_element_type=jnp.float32) + bp_ref[...]


def _final(partials, counts, W_pred, b_pred):
    return pl.pallas_call(
        _final_body,
        out_shape=jax.ShapeDtypeStruct((NUM_GRAPHS, W_pred.shape[1]), jnp.float32),
    )(partials, counts, W_pred, b_pred)


def kernel(x_g, edge_index_g, edge_attr_g, x_lg, edge_index_lg, edge_dist_basis,
           edge_attr_lg, batch, W_enc, b_enc, W_msg, b_msg, W_nb, b_nb, W_eb, b_eb,
           W_mlp, b_mlp, gamma, beta, t, W_pred, b_pred):
    L = W_mlp.shape[0]
    f32 = jnp.float32
    i32 = jnp.int32

    src = edge_index_lg[0].astype(i32)
    dst = edge_index_lg[1].astype(i32)
    order = jnp.argsort(dst)
    dst_s = jnp.take(dst, order)
    src_s = jnp.take(src, order)
    dloc = (dst_s % D_BLK).astype(i32)
    bounds = jnp.arange(0, E_G + 1, D_BLK, dtype=i32)
    boff = jnp.searchsorted(dst_s, bounds).astype(i32)
    boff = jnp.concatenate([boff, jnp.full((BOFF_PAD - boff.shape[0],), E_LG, i32)])
    src_p = jnp.concatenate([src_s, jnp.zeros((EP - E_LG,), i32)])
    dloc_p = jnp.concatenate([dloc, jnp.zeros((EP - E_LG,), i32)])
    ea_lg_p = jnp.concatenate(
        [jnp.take(edge_attr_lg, order, axis=0),
         jnp.zeros((EP - E_LG, edge_attr_lg.shape[1]), f32)], axis=0)

    sg = edge_index_g[0].astype(i32)
    dg = edge_index_g[1].astype(i32)
    ge = jnp.take(batch, dg).astype(i32)
    grid100 = jnp.arange(NUM_GRAPHS + 1, dtype=i32)
    cuts = jnp.searchsorted(batch, grid100)
    counts = jnp.maximum((cuts[1:] - cuts[:-1]).astype(f32), 1.0)

    Wm1 = W_msg[:HID]
    Wm2 = W_msg[HID:2 * HID]
    Wm3 = W_msg[2 * HID:2 * HID + 16]
    Wm4 = W_msg[2 * HID + 16:]
    be = b_enc.reshape(1, HID)
    bm = b_msg.reshape(1, HID)
    bnb = b_nb.reshape(1, HID)
    beb = b_eb.reshape(1, HID)

    A, B = _encoder(x_g, W_enc, be, Wm1, Wm2)
    C, nb = _c_and_nb(edge_attr_g, x_lg, edge_dist_basis, Wm3, Wm4, bm, W_nb, bnb)
    ebs = _edge_basis(ea_lg_p, W_eb, beb)

    h_msg, u = _msg_gather(A, B, C, nb, sg, dg)

    h = None
    x2 = h_msg
    for l in range(L):
        tl = jnp.broadcast_to(t[l], (16,)).astype(f32)
        nd = _edge_pass(u, ebs, src_p, dloc_p, boff, tl)
        h_new, st = _combine(x2, nd, h, W_mlp[l], b_mlp[l].reshape(1, HID))
        h = h_new
        if l < L - 1:
            x2, u = _norm(h, st, nb, gamma[l].reshape(1, HID), beta[l].reshape(1, HID), True)
        else:
            hf = _norm(h, st, nb, gamma[l].reshape(1, HID), beta[l].reshape(1, HID), False)

    partials = _readout(hf, ge)
    return _final(partials, counts.reshape(NUM_GRAPHS, 1), W_pred, b_pred.reshape(1, -1))


# speedup vs baseline: 2.0678x; 1.0811x over previous
import functools

import jax
import jax.numpy as jnp
from jax import lax
from jax.experimental import pallas as pl
from jax.experimental.pallas import tpu as pltpu
from jax.experimental.pallas import tpu_sc as plsc

N_G = 10000
E_G = 160000
E_LG = 480000
HID = 128
NUM_GRAPHS = 100
NREG = HID // 16

NC, NS = 2, 16
NW = NC * NS

D_BLK = 200
NBLK = E_G // D_BLK
BPW = NBLK // NW
E_CHK = 128
EP = 481280
DESC_N = 5376

DLROW = E_CHK + 16
NCH_G = E_G // E_CHK


def _mesh():
    return plsc.VectorSubcoreMesh(core_axis_name="c", subcore_axis_name="s")


def _wid():
    return lax.axis_index("s") * NC + lax.axis_index("c")


def _zero_acc(acc):
    def zero_fn(d2, c2):
        for j in range(2 * NREG):
            acc[d2, pl.ds(j * 16, 16)] = jnp.zeros((16,), jnp.float32)
        return c2
    lax.fori_loop(0, D_BLK, zero_fn, 0)


def _edge_pass_body(u_h, eb_h, sp_h, dl_h, de_h, wo_h, tl_h, nd_h,
                    wo_v, t_v, dbuf, ibuf, dlbuf, ubuf, ebuf, acc,
                    sem_d, sem_i, sem_l, sem_g, sem_e):
    wid = _wid()
    pltpu.sync_copy(wo_h, wo_v)
    pltpu.sync_copy(tl_h, t_v)
    tvec = t_v[...]
    wvec = wo_v[pl.ds(wid, 16)]
    j0 = wvec[0]
    j1 = wvec[1]

    _zero_acc(acc)

    pltpu.sync_copy(de_h.at[j0], dbuf.at[j0 % 4])

    @pl.when(j0 + 1 < j1)
    def _():
        pltpu.sync_copy(de_h.at[j0 + 1], dbuf.at[(j0 + 1) % 4])

    @pl.when(j0 + 2 < j1)
    def _():
        pltpu.async_copy(de_h.at[j0 + 2], dbuf.at[(j0 + 2) % 4], sem_d)

    d0 = dbuf[j0 % 4]
    c00 = pl.multiple_of(d0[0], 8)
    pltpu.sync_copy(sp_h.at[pl.ds(c00, E_CHK)], ibuf.at[j0 % 2])
    pltpu.sync_copy(dl_h.at[pl.ds(c00, E_CHK)], dlbuf.at[pl.ds((j0 % 4) * DLROW, E_CHK)])

    @pl.when(j0 + 1 < j1)
    def _():
        c01 = pl.multiple_of(dbuf[(j0 + 1) % 4][0], 8)
        pltpu.async_copy(sp_h.at[pl.ds(c01, E_CHK)], ibuf.at[(j0 + 1) % 2], sem_i)
        pltpu.async_copy(dl_h.at[pl.ds(c01, E_CHK)], dlbuf.at[pl.ds(((j0 + 1) % 4) * DLROW, E_CHK)], sem_l)

    pltpu.async_copy(u_h.at[ibuf.at[j0 % 2]], ubuf.at[j0 % 2], sem_g)
    pltpu.async_copy(eb_h.at[pl.ds(c00, E_CHK)], ebuf.at[j0 % 2], sem_e)

    def iter_fn(j, cur_b):
        p = j % 2
        r = j % 4

        @pl.when(j + 2 < j1)
        def _():
            pltpu.make_async_copy(de_h.at[0], dbuf.at[0], sem_d).wait()

        @pl.when(j + 3 < j1)
        def _():
            pltpu.async_copy(de_h.at[j + 3], dbuf.at[(j + 3) % 4], sem_d)

        @pl.when(j + 1 < j1)
        def _():
            pltpu.make_async_copy(sp_h.at[pl.ds(0, E_CHK)], ibuf.at[0], sem_i).wait()
            pltpu.make_async_copy(dl_h.at[pl.ds(0, E_CHK)],
                                  dlbuf.at[pl.ds(0, E_CHK)], sem_l).wait()

        pltpu.make_async_copy(u_h.at[pl.ds(0, E_CHK)], ubuf.at[0], sem_g).wait()
        pltpu.make_async_copy(eb_h.at[pl.ds(0, E_CHK)], ebuf.at[0], sem_e).wait()

        @pl.when(j + 1 < j1)
        def _():
            cN = pl.multiple_of(dbuf[(j + 1) % 4][0], 8)
            pltpu.async_copy(u_h.at[ibuf.at[(j + 1) % 2]], ubuf.at[(j + 1) % 2], sem_g)
            pltpu.async_copy(eb_h.at[pl.ds(cN, E_CHK)], ebuf.at[(j + 1) % 2], sem_e)

        @pl.when(j + 2 < j1)
        def _():
            c2 = pl.multiple_of(dbuf[(j + 2) % 4][0], 8)
            pltpu.async_copy(sp_h.at[pl.ds(c2, E_CHK)], ibuf.at[(j + 2) % 2], sem_i)
            pltpu.async_copy(dl_h.at[pl.ds(c2, E_CHK)],
                             dlbuf.at[pl.ds(((j + 2) % 4) * DLROW, E_CHK)], sem_l)

        dvec = dbuf[r]
        b_j = dvec[1]
        lo = dvec[2]
        hi = dvec[3]

        @pl.when(jnp.logical_and(b_j != cur_b, cur_b >= 0))
        def _():
            pltpu.sync_copy(acc, nd_h.at[pl.ds(cur_b * D_BLK, D_BLK)])
            _zero_acc(acc)

        def edge_fn(e, c3):
            d = dlbuf[pl.ds(r * DLROW + e, 16)][0]
            for rr in range(NREG):
                uv = ubuf[p, e, pl.ds(rr * 16, 16)]
                ev = ebuf[p, e, pl.ds(rr * 16, 16)]
                m = jnp.maximum(uv + ev, 0.0) + 1e-7
                ex = jnp.exp(m * tvec)
                plsc.addupdate(acc.at[d, pl.ds(rr * 16, 16)], m * ex)
                plsc.addupdate(acc.at[d, pl.ds(HID + rr * 16, 16)], ex)
            return c3
        lax.fori_loop(lo, hi, edge_fn, 0)
        return b_j

    last_b = lax.fori_loop(j0, j1, iter_fn, -1)
    pltpu.sync_copy(acc, nd_h.at[pl.ds(last_b * D_BLK, D_BLK)])


@functools.partial(
    pl.kernel,
    out_type=jax.ShapeDtypeStruct((E_G, 2 * HID), jnp.float32),
    mesh=_mesh(),
    scratch_types=[
        pltpu.VMEM((48,), jnp.int32),
        pltpu.VMEM((16,), jnp.float32),
        pltpu.VMEM((4, 16), jnp.int32),
        pltpu.VMEM((2, E_CHK), jnp.int32),
        pltpu.VMEM((4 * DLROW,), jnp.int32),
        pltpu.VMEM((2, E_CHK, HID), jnp.float32),
        pltpu.VMEM((2, E_CHK, HID), jnp.float32),
        pltpu.VMEM((D_BLK, 2 * HID), jnp.float32),
        pltpu.SemaphoreType.DMA,
        pltpu.SemaphoreType.DMA,
        pltpu.SemaphoreType.DMA,
        pltpu.SemaphoreType.DMA,
        pltpu.SemaphoreType.DMA,
    ],
)
def _edge_pass(*refs):
    _edge_pass_body(*refs)


def _msg_body(a_h, b_h, c_h, nb_h, sg_h, dg_h, hm_h, u0_h,
              sgv, dgv, av, bv, cv, nv, s1, s2, s3, s4):
    wid = _wid()

    def chunk_fn(k, carry):
        ch = wid + NW * k

        @pl.when(ch < NCH_G)
        def _():
            base = ch * E_CHK
            pltpu.sync_copy(sg_h.at[pl.ds(base, E_CHK)], sgv)
            pltpu.sync_copy(dg_h.at[pl.ds(base, E_CHK)], dgv)
            cp1 = pltpu.async_copy(a_h.at[sgv], av, s1)
            cp2 = pltpu.async_copy(b_h.at[dgv], bv, s2)
            cp3 = pltpu.async_copy(c_h.at[pl.ds(base, E_CHK)], cv, s3)
            cp4 = pltpu.async_copy(nb_h.at[pl.ds(base, E_CHK)], nv, s4)
            cp1.wait()
            cp2.wait()
            cp3.wait()
            cp4.wait()

            def edge_fn(e, c3):
                for r in range(NREG):
                    sl = pl.ds(r * 16, 16)
                    hm = av[e, sl] + bv[e, sl] + cv[e, sl]
                    av[e, sl] = hm
                    cv[e, sl] = hm * nv[e, sl]
                return c3
            lax.fori_loop(0, E_CHK, edge_fn, 0)
            pltpu.sync_copy(av, hm_h.at[pl.ds(base, E_CHK)])
            pltpu.sync_copy(cv, u0_h.at[pl.ds(base, E_CHK)])
        return carry
    lax.fori_loop(0, (NCH_G + NW - 1) // NW, chunk_fn, 0)


@functools.partial(
    pl.kernel,
    out_type=[jax.ShapeDtypeStruct((E_G, HID), jnp.float32),
              jax.ShapeDtypeStruct((E_G, HID), jnp.float32)],
    mesh=_mesh(),
    scratch_types=[
        pltpu.VMEM((E_CHK,), jnp.int32),
        pltpu.VMEM((E_CHK,), jnp.int32),
        pltpu.VMEM((E_CHK, HID), jnp.float32),
        pltpu.VMEM((E_CHK, HID), jnp.float32),
        pltpu.VMEM((E_CHK, HID), jnp.float32),
        pltpu.VMEM((E_CHK, HID), jnp.float32),
        pltpu.SemaphoreType.DMA,
        pltpu.SemaphoreType.DMA,
        pltpu.SemaphoreType.DMA,
        pltpu.SemaphoreType.DMA,
    ],
)
def _msg_gather(*refs):
    _msg_body(*refs)


def _readout_body(hf_h, ge_h, part_h, gev, hv, acc, s1):
    wid = _wid()

    def zero_fn(d, c2):
        for j in range(NREG):
            acc[d, pl.ds(j * 16, 16)] = jnp.zeros((16,), jnp.float32)
        return c2
    lax.fori_loop(0, NUM_GRAPHS, zero_fn, 0)

    def chunk_fn(k, carry):
        ch = wid + NW * k

        @pl.when(ch < NCH_G)
        def _():
            base = ch * E_CHK
            pltpu.sync_copy(ge_h.at[pl.ds(base, E_CHK)], gev.at[pl.ds(0, E_CHK)])
            pltpu.sync_copy(hf_h.at[pl.ds(base, E_CHK)], hv)

            def edge_fn(e, c3):
                g = gev[pl.ds(e, 16)][0]
                for r in range(NREG):
                    plsc.addupdate(acc.at[g, pl.ds(r * 16, 16)], hv[e, pl.ds(r * 16, 16)])
                return c3
            lax.fori_loop(0, E_CHK, edge_fn, 0)
        return carry
    lax.fori_loop(0, (NCH_G + NW - 1) // NW, chunk_fn, 0)
    pltpu.sync_copy(acc, part_h.at[wid])


@functools.partial(
    pl.kernel,
    out_type=jax.ShapeDtypeStruct((NW, NUM_GRAPHS, HID), jnp.float32),
    mesh=_mesh(),
    scratch_types=[
        pltpu.VMEM((E_CHK + 16,), jnp.int32),
        pltpu.VMEM((E_CHK, HID), jnp.float32),
        pltpu.VMEM((NUM_GRAPHS, HID), jnp.float32),
        pltpu.SemaphoreType.DMA,
    ],
)
def _readout(*refs):
    _readout_body(*refs)


def _enc_body(x_ref, we_ref, be_ref, w1_ref, w2_ref, a_ref, b_ref):
    h0 = jnp.dot(x_ref[...], we_ref[...], preferred_element_type=jnp.float32) + be_ref[...]
    a_ref[...] = jnp.dot(h0, w1_ref[...], preferred_element_type=jnp.float32)
    b_ref[...] = jnp.dot(h0, w2_ref[...], preferred_element_type=jnp.float32)


def _encoder(x_g, W_enc, b_enc, Wm1, Wm2):
    return pl.pallas_call(
        _enc_body,
        out_shape=[jax.ShapeDtypeStruct((N_G, HID), jnp.float32),
                   jax.ShapeDtypeStruct((N_G, HID), jnp.float32)],
    )(x_g, W_enc, b_enc, Wm1, Wm2)


def _smallk_mm(x_ref, w_ref, k):
    out = x_ref[:, 0:1] * w_ref[0:1, :]
    for j in range(1, k):
        out = out + x_ref[:, j:j + 1] * w_ref[j:j + 1, :]
    return out


def _cnb_body(ea_ref, xlg_ref, edb_ref, w3_ref, w4_ref, bm_ref, wnb_ref, bnb_ref,
              c_ref, nb_ref):
    c_ref[...] = (_smallk_mm(ea_ref, w3_ref, 16) + _smallk_mm(xlg_ref, w4_ref, 4)
                  + bm_ref[...])
    nb_ref[...] = _smallk_mm(edb_ref, wnb_ref, 4) + bnb_ref[...]


def _c_and_nb(ea_g, x_lg, edb, Wm3, Wm4, b_msg, W_nb, b_nb):
    R = 4000
    grid = (E_G // R,)
    return pl.pallas_call(
        _cnb_body,
        grid=grid,
        in_specs=[
            pl.BlockSpec((R, 16), lambda i: (i, 0)),
            pl.BlockSpec((R, 4), lambda i: (i, 0)),
            pl.BlockSpec((R, 4), lambda i: (i, 0)),
            pl.BlockSpec((16, HID), lambda i: (0, 0)),
            pl.BlockSpec((4, HID), lambda i: (0, 0)),
            pl.BlockSpec((1, HID), lambda i: (0, 0)),
            pl.BlockSpec((4, HID), lambda i: (0, 0)),
            pl.BlockSpec((1, HID), lambda i: (0, 0)),
        ],
        out_specs=[pl.BlockSpec((R, HID), lambda i: (i, 0)),
                   pl.BlockSpec((R, HID), lambda i: (i, 0))],
        out_shape=[jax.ShapeDtypeStruct((E_G, HID), jnp.float32),
                   jax.ShapeDtypeStruct((E_G, HID), jnp.float32)],
    )(ea_g, x_lg, edb, Wm3, Wm4, b_msg, W_nb, b_nb)


def _eb_body(ea_ref, web_ref, beb_ref, o_ref):
    o_ref[...] = _smallk_mm(ea_ref, web_ref, 4) + beb_ref[...]


def _edge_basis(ea_lg_p, W_eb, b_eb):
    R = 2560
    assert EP % R == 0
    grid = (EP // R,)
    return pl.pallas_call(
        _eb_body,
        grid=grid,
        in_specs=[
            pl.BlockSpec((R, 4), lambda i: (i, 0)),
            pl.BlockSpec((4, HID), lambda i: (0, 0)),
            pl.BlockSpec((1, HID), lambda i: (0, 0)),
        ],
        out_specs=pl.BlockSpec((R, HID), lambda i: (i, 0)),
        out_shape=jax.ShapeDtypeStruct((EP, HID), jnp.float32),
    )(ea_lg_p, W_eb, b_eb)


def _combine_body(has_res, nsteps, x2_ref, nd_ref, res_ref, w_ref, b_ref,
                  h_ref, st_ref, acc_ref):
    i = pl.program_id(0)
    num = nd_ref[:, :HID]
    den = nd_ref[:, HID:]
    xa = x2_ref[...] + num / (den + 1e-16)
    h = jnp.dot(xa, w_ref[...], preferred_element_type=jnp.float32) + b_ref[...]
    if has_res:
        h = h + res_ref[...]
    h_ref[...] = h

    @pl.when(i == 0)
    def _():
        acc_ref[...] = jnp.zeros_like(acc_ref)
    s = jnp.sum(h, axis=0, keepdims=True)
    s2 = jnp.sum(h * h, axis=0, keepdims=True)
    acc_ref[0:1, :] += s
    acc_ref[1:2, :] += s2

    @pl.when(i == nsteps - 1)
    def _():
        st_ref[...] = acc_ref[...]


def _combine(x2, nd, res, W, b):
    R = 2000
    nsteps = E_G // R
    has_res = res is not None
    in_specs = [
        pl.BlockSpec((R, HID), lambda i: (i, 0)),
        pl.BlockSpec((R, 2 * HID), lambda i: (i, 0)),
    ]
    args = [x2, nd]
    if has_res:
        in_specs.append(pl.BlockSpec((R, HID), lambda i: (i, 0)))
        args.append(res)
    in_specs += [pl.BlockSpec((HID, HID), lambda i: (0, 0)),
                 pl.BlockSpec((1, HID), lambda i: (0, 0))]
    args += [W, b]

    def body(*refs):
        if has_res:
            x2_ref, nd_ref, res_ref, w_ref, b_ref, h_ref, st_ref, acc_ref = refs
        else:
            x2_ref, nd_ref, w_ref, b_ref, h_ref, st_ref, acc_ref = refs
            res_ref = None
        _combine_body(has_res, nsteps, x2_ref, nd_ref, res_ref, w_ref, b_ref,
                      h_ref, st_ref, acc_ref)

    return pl.pallas_call(
        body,
        grid=(nsteps,),
        in_specs=in_specs,
        out_specs=[pl.BlockSpec((R, HID), lambda i: (i, 0)),
                   pl.BlockSpec((8, HID), lambda i: (0, 0))],
        out_shape=[jax.ShapeDtypeStruct((E_G, HID), jnp.float32),
                   jax.ShapeDtypeStruct((8, HID), jnp.float32)],
        scratch_shapes=[pltpu.VMEM((8, HID), jnp.float32)],
    )(*args)


def _norm_body(with_relu_u, h_ref, st_ref, nb_ref, g_ref, bt_ref, h2_ref, u_ref):
    mu = st_ref[0:1, :] / E_G
    var = st_ref[1:2, :] / E_G - mu * mu
    inv = lax.rsqrt(var + 1e-5)
    xn = (h_ref[...] - mu) * inv * g_ref[...] + bt_ref[...]
    if with_relu_u:
        h2 = jnp.maximum(xn, 0.0)
        h2_ref[...] = h2
        u_ref[...] = h2 * nb_ref[...]
    else:
        h2_ref[...] = xn


def _norm(h, st, nb, g, bt, with_relu_u):
    R = 2000
    nsteps = E_G // R
    in_specs = [
        pl.BlockSpec((R, HID), lambda i: (i, 0)),
        pl.BlockSpec((8, HID), lambda i: (0, 0)),
        pl.BlockSpec((R, HID), lambda i: (i, 0)),
        pl.BlockSpec((1, HID), lambda i: (0, 0)),
        pl.BlockSpec((1, HID), lambda i: (0, 0)),
    ]
    if with_relu_u:
        out_specs = [pl.BlockSpec((R, HID), lambda i: (i, 0)),
                     pl.BlockSpec((R, HID), lambda i: (i, 0))]
        out_shape = [jax.ShapeDtypeStruct((E_G, HID), jnp.float32),
                     jax.ShapeDtypeStruct((E_G, HID), jnp.float32)]
    else:
        out_specs = [pl.BlockSpec((R, HID), lambda i: (i, 0))]
        out_shape = [jax.ShapeDtypeStruct((E_G, HID), jnp.float32)]

    def body(*refs):
        if with_relu_u:
            h_ref, st_ref, nb_ref, g_ref, bt_ref, h2_ref, u_ref = refs
        else:
            h_ref, st_ref, nb_ref, g_ref, bt_ref, h2_ref = refs
            u_ref = None
        _norm_body(with_relu_u, h_ref, st_ref, nb_ref, g_ref, bt_ref, h2_ref, u_ref)

    res = pl.pallas_call(
        body,
        grid=(nsteps,),
        in_specs=in_specs,
        out_specs=out_specs,
        out_shape=out_shape,
    )(h, st, nb, g, bt)
    return res if with_relu_u else res[0]


def _final_body(p_ref, cnt_ref, wp_ref, bp_ref, o_ref):
    s = p_ref[0]
    for w in range(1, NW):
        s = s + p_ref[w]
    hg = s / cnt_ref[...]
    o_ref[...] = jnp.dot(hg, wp_ref[...], preferred_element_type=jnp.float32) + bp_ref[...]


def _final(partials, counts, W_pred, b_pred):
    return pl.pallas_call(
        _final_body,
        out_shape=jax.ShapeDtypeStruct((NUM_GRAPHS, W_pred.shape[1]), jnp.float32),
    )(partials, counts, W_pred, b_pred)


def kernel(x_g, edge_index_g, edge_attr_g, x_lg, edge_index_lg, edge_dist_basis,
           edge_attr_lg, batch, W_enc, b_enc, W_msg, b_msg, W_nb, b_nb, W_eb, b_eb,
           W_mlp, b_mlp, gamma, beta, t, W_pred, b_pred):
    L = W_mlp.shape[0]
    f32 = jnp.float32
    i32 = jnp.int32

    src = edge_index_lg[0].astype(i32)
    dst = edge_index_lg[1].astype(i32)
    order = jnp.argsort(dst)
    dst_s = jnp.take(dst, order)
    src_s = jnp.take(src, order)
    dloc = (dst_s % D_BLK).astype(i32)
    bounds = jnp.arange(0, E_G + 1, D_BLK, dtype=i32)
    boff = jnp.searchsorted(dst_s, bounds).astype(i32)
    c0s = (boff[:-1] // 8) * 8
    nchb = jnp.maximum((boff[1:] - c0s + E_CHK - 1) // E_CHK, 1)
    ends = jnp.cumsum(nchb).astype(i32)
    starts = ends - nchb
    jj = jnp.arange(DESC_N, dtype=i32)
    bj = jnp.clip(jnp.searchsorted(ends, jj, side="right"), 0, NBLK - 1).astype(i32)
    ii = jj - starts[bj]
    cj = c0s[bj] + E_CHK * ii
    loj = jnp.clip(boff[bj] - cj, 0, E_CHK)
    hij = jnp.clip(boff[bj + 1] - cj, 0, E_CHK)
    cj = jnp.clip(cj, 0, EP - E_CHK)
    zc = jnp.zeros_like(cj)
    desc = jnp.stack([cj, bj, loj, hij] + [zc] * 12, axis=1)
    wdo = jnp.concatenate([jnp.zeros((1,), i32), ends[BPW - 1::BPW]])
    wdo = jnp.concatenate([wdo, jnp.full((48 - NW - 1,), wdo[-1], i32)])
    src_p = jnp.concatenate([src_s, jnp.zeros((EP - E_LG,), i32)])
    dloc_p = jnp.concatenate([dloc, jnp.zeros((EP - E_LG,), i32)])
    ea_lg_p = jnp.concatenate(
        [jnp.take(edge_attr_lg, order, axis=0),
         jnp.zeros((EP - E_LG, edge_attr_lg.shape[1]), f32)], axis=0)

    sg = edge_index_g[0].astype(i32)
    dg = edge_index_g[1].astype(i32)
    ge = jnp.take(batch, dg).astype(i32)
    grid100 = jnp.arange(NUM_GRAPHS + 1, dtype=i32)
    cuts = jnp.searchsorted(batch, grid100)
    counts = jnp.maximum((cuts[1:] - cuts[:-1]).astype(f32), 1.0)

    Wm1 = W_msg[:HID]
    Wm2 = W_msg[HID:2 * HID]
    Wm3 = W_msg[2 * HID:2 * HID + 16]
    Wm4 = W_msg[2 * HID + 16:]
    be = b_enc.reshape(1, HID)
    bm = b_msg.reshape(1, HID)
    bnb = b_nb.reshape(1, HID)
    beb = b_eb.reshape(1, HID)

    A, B = _encoder(x_g, W_enc, be, Wm1, Wm2)
    C, nb = _c_and_nb(edge_attr_g, x_lg, edge_dist_basis, Wm3, Wm4, bm, W_nb, bnb)
    ebs = _edge_basis(ea_lg_p, W_eb, beb)

    h_msg, u = _msg_gather(A, B, C, nb, sg, dg)

    h = None
    x2 = h_msg
    for l in range(L):
        tl = jnp.broadcast_to(t[l], (16,)).astype(f32)
        nd = _edge_pass(u, ebs, src_p, dloc_p, desc, wdo, tl)
        h_new, st = _combine(x2, nd, h, W_mlp[l], b_mlp[l].reshape(1, HID))
        h = h_new
        if l < L - 1:
            x2, u = _norm(h, st, nb, gamma[l].reshape(1, HID), beta[l].reshape(1, HID), True)
        else:
            hf = _norm(h, st, nb, gamma[l].reshape(1, HID), beta[l].reshape(1, HID), False)

    partials = _readout(hf, ge)
    return _final(partials, counts.reshape(NUM_GRAPHS, 1), W_pred, b_pred.reshape(1, -1))
